# baseline (speedup 1.0000x reference)
import functools

import jax
import jax.numpy as jnp
from jax import lax
from jax.experimental import pallas as pl
from jax.experimental.pallas import tpu as pltpu
from jax.experimental.pallas import tpu_sc as plsc

BATCH = 16384
EMB = 8
ROWS_PER_SUPER = 128 // EMB
NC = 2
NS = 16
NW = NC * NS
BPW = BATCH // NW
CHUNK = 128
NCHUNK = BPW // CHUNK
IDX_ROWS = BATCH // CHUNK


def _sc_gather(usid, isid, t_um, t_im, t_uf, t_if):
    mesh = plsc.VectorSubcoreMesh(core_axis_name="c", subcore_axis_name="s")
    out_t = [jax.ShapeDtypeStruct((BATCH, 128), jnp.float32)] * 4

    @functools.partial(
        pl.kernel,
        mesh=mesh,
        out_type=out_t,
        scratch_types=[
            pltpu.VMEM((NCHUNK, CHUNK), jnp.int32),
            pltpu.VMEM((NCHUNK, CHUNK), jnp.int32),
            pltpu.VMEM((2, CHUNK, 128), jnp.float32),
            pltpu.SemaphoreType.DMA,
            pltpu.SemaphoreType.DMA,
        ],
    )
    def k(u_hbm, i_hbm, um_hbm, im_hbm, uf_hbm, if_hbm,
          o_um, o_im, o_uf, o_if,
          uidx, iidx, gbuf, gsem, wsem):
        wid = lax.axis_index("s") * NC + lax.axis_index("c")
        base = wid * BPW
        row0 = wid * NCHUNK
        pltpu.sync_copy(u_hbm.at[pl.ds(row0, NCHUNK)], uidx)
        pltpu.sync_copy(i_hbm.at[pl.ds(row0, NCHUNK)], iidx)

        units = []
        for t_hbm, o_hbm, idxb in ((um_hbm, o_um, uidx), (im_hbm, o_im, iidx),
                                   (uf_hbm, o_uf, uidx), (if_hbm, o_if, iidx)):
            for c in range(NCHUNK):
                units.append((t_hbm, o_hbm, idxb, c))

        def fire(i):
            t_hbm, _, idxb, c = units[i]
            return pltpu.async_copy(t_hbm.at[idxb.at[c]], gbuf.at[i % 2], gsem)

        gathers = [None] * len(units)
        writes = [None] * len(units)
        gathers[0] = fire(0)
        for i in range(len(units)):
            if i + 1 < len(units):
                if i >= 1:
                    writes[i - 1].wait()
                gathers[i + 1] = fire(i + 1)
            gathers[i].wait()
            _, o_hbm, _, c = units[i]
            writes[i] = pltpu.async_copy(
                gbuf.at[i % 2], o_hbm.at[pl.ds(base + c * CHUNK, CHUNK)], wsem)
        writes[-2].wait()
        writes[-1].wait()

    return k(usid, isid, t_um, t_im, t_uf, t_if)


BT = 512


def _tc_body(gum, gim, guf, gif, uoff, ioff,
             w1u, w1i, b1r, w2, b2r, wah, waf, bar, out):
    uo = uoff[...]
    io = ioff[...]

    def extract(g, off):
        acc = jnp.zeros((BT, EMB), jnp.float32)
        for s in range(ROWS_PER_SUPER):
            acc = jnp.where(off == s, g[:, s * EMB:(s + 1) * EMB], acc)
        return acc

    um = extract(gum[...], uo)
    im = extract(gim[...], io)
    uf = extract(guf[...], uo)
    itf = extract(gif[...], io)
    h = jnp.maximum(
        jnp.dot(um, w1u[...], preferred_element_type=jnp.float32)
        + jnp.dot(im, w1i[...], preferred_element_type=jnp.float32)
        + b1r[...], 0.0)
    h2 = jnp.maximum(
        jnp.dot(h, w2[...], preferred_element_type=jnp.float32) + b2r[...], 0.0)
    mf = uf * itf
    logits = (jnp.dot(h2, wah[...], preferred_element_type=jnp.float32)
              + jnp.dot(mf, waf[...], preferred_element_type=jnp.float32)
              + bar[...])
    out[...] = jax.nn.sigmoid(logits)


def _tc_dense(gum, gim, guf, gif, uoff, ioff,
              w1u, w1i, b1r, w2, b2r, wah, waf, bar):
    grid = BATCH // BT
    g_spec = pl.BlockSpec((BT, 128), lambda i: (i, 0))
    o_spec = pl.BlockSpec((BT, 1), lambda i: (i, 0))

    def wspec(shape):
        return pl.BlockSpec(shape, lambda i: (0, 0))

    return pl.pallas_call(
        _tc_body,
        grid=(grid,),
        in_specs=[
            g_spec, g_spec, g_spec, g_spec, o_spec, o_spec,
            wspec((EMB, 32)), wspec((EMB, 32)), wspec((1, 32)),
            wspec((32, EMB)), wspec((1, EMB)),
            wspec((EMB, 1)), wspec((EMB, 1)), wspec((1, 1)),
        ],
        out_specs=pl.BlockSpec((BT, 1), lambda i: (i, 0)),
        out_shape=jax.ShapeDtypeStruct((BATCH, 1), jnp.float32),
    )(gum, gim, guf, gif, uoff, ioff,
      w1u, w1i, b1r, w2, b2r, wah, waf, bar)


def kernel(user_input, item_input, emb_user_mlp, emb_item_mlp,
           emb_user_mf, emb_item_mf, W1, b1, W2, b2, Wa, ba):
    ui = user_input.astype(jnp.int32)
    ii = item_input.astype(jnp.int32)
    usid = (ui // ROWS_PER_SUPER).reshape(IDX_ROWS, CHUNK)
    isid = (ii // ROWS_PER_SUPER).reshape(IDX_ROWS, CHUNK)
    uoff = (ui % ROWS_PER_SUPER).reshape(BATCH, 1)
    ioff = (ii % ROWS_PER_SUPER).reshape(BATCH, 1)
    gum, gim, guf, gif = _sc_gather(
        usid, isid,
        emb_user_mlp.reshape(-1, 128), emb_item_mlp.reshape(-1, 128),
        emb_user_mf.reshape(-1, 128), emb_item_mf.reshape(-1, 128))
    w1u, w1i = W1[:EMB], W1[EMB:]
    wah, waf = Wa[:EMB], Wa[EMB:]
    return _tc_dense(
        gum, gim, guf, gif, uoff, ioff,
        w1u, w1i, b1.reshape(1, 32),
        W2, b2.reshape(1, EMB),
        wah, waf, ba.reshape(1, 1))


# speedup vs baseline: 12.8563x; 12.8563x over previous
import functools

import jax
import jax.numpy as jnp
from jax import lax
from jax.experimental import pallas as pl
from jax.experimental.pallas import tpu as pltpu
from jax.experimental.pallas import tpu_sc as plsc

BATCH = 16384
EMB = 8
NUM = 1000000
FSTRIDE = 1 << 20
NC = 2
NS = 16
NW = NC * NS
BPW = BATCH // NW
CHUNK = 128
NCHUNK = BPW // CHUNK
IDX_ROWS = BATCH // CHUNK

UK = 16
UB = FSTRIDE // UK


def _untile_body(*refs):
    ins = refs[:4]
    outs = refs[4:]
    for t in range(4):
        for f in range(EMB):
            outs[t * EMB + f][...] = ins[t][f, :]


def _tc_untile(ta, tb, tc_, td):
    in_spec = pl.BlockSpec((EMB, UB), lambda k: (0, k))
    out_spec = pl.BlockSpec((UB,), lambda k: (k,))
    return pl.pallas_call(
        _untile_body,
        grid=(UK,),
        in_specs=[in_spec] * 4,
        out_specs=[out_spec] * 32,
        out_shape=[jax.ShapeDtypeStruct((FSTRIDE,), jnp.float32)] * 32,
    )(ta, tb, tc_, td)


def _sc_gather(u2d, i2d, feats):
    mesh = plsc.VectorSubcoreMesh(core_axis_name="c", subcore_axis_name="s")
    out_t = [jax.ShapeDtypeStruct((EMB, BATCH), jnp.float32)] * 4

    @functools.partial(
        pl.kernel,
        mesh=mesh,
        out_type=out_t,
        compiler_params=pltpu.CompilerParams(use_tc_tiling_on_sc=False),
        scratch_types=[
            pltpu.VMEM((NCHUNK, CHUNK), jnp.int32),
            pltpu.VMEM((NCHUNK, CHUNK), jnp.int32),
            pltpu.VMEM((EMB, BPW), jnp.float32),
            pltpu.VMEM((EMB, BPW), jnp.float32),
            pltpu.VMEM((EMB, BPW), jnp.float32),
            pltpu.VMEM((EMB, BPW), jnp.float32),
            pltpu.SemaphoreType.DMA,
        ],
    )
    def k(u_hbm, i_hbm, *rest):
        s = rest[:32]
        o_um, o_im, o_uf, o_if = rest[32:36]
        uidx, iidx, r_um, r_im, r_uf, r_if, sem = rest[36:]
        wid = lax.axis_index("s") * NC + lax.axis_index("c")
        base = wid * BPW
        row0 = wid * NCHUNK
        pltpu.sync_copy(u_hbm.at[pl.ds(row0, NCHUNK)], uidx)
        pltpu.sync_copy(i_hbm.at[pl.ds(row0, NCHUNK)], iidx)
        copies = []
        for t, (rbuf, idxb) in enumerate(
                ((r_um, uidx), (r_im, iidx), (r_uf, uidx), (r_if, iidx))):
            for f in range(EMB):
                src = s[t * EMB + f]
                for g in range(NCHUNK):
                    copies.append(pltpu.async_copy(
                        src.at[idxb.at[g]],
                        rbuf.at[f, pl.ds(g * CHUNK, CHUNK)], sem))
        for c in copies:
            c.wait()
        osl = pl.ds(base, BPW)
        pltpu.sync_copy(r_um, o_um.at[:, osl])
        pltpu.sync_copy(r_im, o_im.at[:, osl])
        pltpu.sync_copy(r_uf, o_uf.at[:, osl])
        pltpu.sync_copy(r_if, o_if.at[:, osl])

    return k(u2d, i2d, *feats)


BT = 2048


def _tc_body(um, im, uf, itf, w1u, w1i, b1c, w2, b2c, wah, waf, bac, out):
    h = jnp.maximum(
        jnp.dot(w1u[...], um[...], preferred_element_type=jnp.float32)
        + jnp.dot(w1i[...], im[...], preferred_element_type=jnp.float32)
        + b1c[...], 0.0)
    h2 = jnp.maximum(
        jnp.dot(w2[...], h, preferred_element_type=jnp.float32) + b2c[...], 0.0)
    mf = uf[...] * itf[...]
    logits = (jnp.dot(wah[...], h2, preferred_element_type=jnp.float32)
              + jnp.dot(waf[...], mf, preferred_element_type=jnp.float32)
              + bac[...])
    out[...] = jax.nn.sigmoid(logits)


def _tc_dense(u_mlp, i_mlp, u_mf, i_mf, w1u, w1i, b1c, w2, b2c, wah, waf, bac):
    grid = BATCH // BT
    emb_spec = pl.BlockSpec((EMB, BT), lambda i: (0, i))

    def wspec(shape):
        return pl.BlockSpec(shape, lambda i: (0, 0))

    return pl.pallas_call(
        _tc_body,
        grid=(grid,),
        in_specs=[
            emb_spec, emb_spec, emb_spec, emb_spec,
            wspec((32, EMB)), wspec((32, EMB)), wspec((32, 1)),
            wspec((EMB, 32)), wspec((EMB, 1)),
            wspec((1, EMB)), wspec((1, EMB)), wspec((1, 1)),
        ],
        out_specs=pl.BlockSpec((1, BT), lambda i: (0, i)),
        out_shape=jax.ShapeDtypeStruct((1, BATCH), jnp.float32),
    )(u_mlp, i_mlp, u_mf, i_mf, w1u, w1i, b1c, w2, b2c, wah, waf, bac)


def kernel(user_input, item_input, emb_user_mlp, emb_item_mlp,
           emb_user_mf, emb_item_mf, W1, b1, W2, b2, Wa, ba):
    u2d = user_input.astype(jnp.int32).reshape(IDX_ROWS, CHUNK)
    i2d = item_input.astype(jnp.int32).reshape(IDX_ROWS, CHUNK)
    feats = _tc_untile(emb_user_mlp.T, emb_item_mlp.T,
                       emb_user_mf.T, emb_item_mf.T)
    gum, gim, guf, gif = _sc_gather(u2d, i2d, feats)
    out_t = _tc_dense(
        gum, gim, guf, gif,
        W1[:EMB].T, W1[EMB:].T, b1.reshape(32, 1),
        W2.T, b2.reshape(EMB, 1),
        Wa[:EMB].T, Wa[EMB:].T, ba.reshape(1, 1))
    return out_t.reshape(BATCH, 1)
